# speedup vs baseline: 1.4086x; 1.4086x over previous
import functools

import jax
import jax.numpy as jnp
from jax import lax
from jax.experimental import pallas as pl
from jax.experimental.pallas import tpu as pltpu

N = 16384
DIM = 2048
E = 64
K = 8
BN = 512


def _router_body(x_ref, bias_ref, w_ref, scores_ref, tops_ref, idx_ref,
                 counts_ref, ent_ref):
    i = pl.program_id(0)
    nsteps = pl.num_programs(0)

    logits = lax.dot_general(
        x_ref[...], w_ref[...], (((1,), (1,)), ((), ())),
        preferred_element_type=jnp.float32)
    scores = jax.nn.sigmoid(logits)
    scores_ref[...] = scores

    biased = scores + bias_ref[...]
    iota = lax.broadcasted_iota(jnp.int32, (BN, E), 1)

    work = biased
    idx_cols = []
    raw_cols = []
    counts = jnp.zeros((1, E), jnp.float32)
    for _ in range(K):
        m = jnp.max(work, axis=1, keepdims=True)
        is_max = work == m
        idx = jnp.min(jnp.where(is_max, iota, E), axis=1, keepdims=True)
        onehot = iota == idx
        raw = jnp.sum(jnp.where(onehot, scores, 0.0), axis=1, keepdims=True)
        idx_cols.append(idx)
        raw_cols.append(raw)
        counts = counts + jnp.sum(onehot.astype(jnp.float32), axis=0,
                                  keepdims=True)
        work = jnp.where(onehot, -jnp.inf, work)

    idx_blk = jnp.concatenate(idx_cols, axis=1)
    raw_blk = jnp.concatenate(raw_cols, axis=1)
    denom = jnp.sum(raw_blk, axis=1, keepdims=True) + 1e-20
    tops = raw_blk / denom
    idx_ref[...] = idx_blk
    tops_ref[...] = tops

    ent_part = jnp.sum(tops * jnp.log(tops))

    @pl.when(i == 0)
    def _init():
        counts_ref[...] = counts
        ent_ref[...] = jnp.full((1, 1), ent_part, jnp.float32)

    @pl.when(i > 0)
    def _acc():
        counts_ref[...] += counts
        ent_ref[...] += ent_part

    @pl.when(i == nsteps - 1)
    def _fin():
        ent_ref[...] = -ent_ref[...] * (1.0 / N)


@jax.jit
def kernel(x, expert_bias, W):
    grid = (N // BN,)
    scores, tops, idx, counts, ent = pl.pallas_call(
        _router_body,
        grid=grid,
        in_specs=[
            pl.BlockSpec((BN, DIM), lambda i: (i, 0)),
            pl.BlockSpec((1, E), lambda i: (0, 0)),
            pl.BlockSpec((E, DIM), lambda i: (0, 0)),
        ],
        out_specs=[
            pl.BlockSpec((BN, E), lambda i: (i, 0)),
            pl.BlockSpec((BN, K), lambda i: (i, 0)),
            pl.BlockSpec((BN, K), lambda i: (i, 0)),
            pl.BlockSpec((1, E), lambda i: (0, 0)),
            pl.BlockSpec((1, 1), lambda i: (0, 0)),
        ],
        out_shape=[
            jax.ShapeDtypeStruct((N, E), jnp.float32),
            jax.ShapeDtypeStruct((N, K), jnp.float32),
            jax.ShapeDtypeStruct((N, K), jnp.int32),
            jax.ShapeDtypeStruct((1, E), jnp.float32),
            jax.ShapeDtypeStruct((1, 1), jnp.float32),
        ],
    )(x, expert_bias.reshape(1, E), W)
    return (tops, scores, idx, counts.reshape(E), ent.reshape(()))


# speedup vs baseline: 2.0444x; 1.4513x over previous
import functools

import jax
import jax.numpy as jnp
from jax import lax
from jax.experimental import pallas as pl
from jax.experimental.pallas import tpu as pltpu

N = 16384
DIM = 2048
E = 64
K = 8
BN = 512


def _router_body(x_ref, bias_ref, w_ref, scores_ref, tops_ref, idx_ref,
                 counts_ref, ent_ref):
    i = pl.program_id(0)
    nsteps = pl.num_programs(0)

    logits_t = lax.dot_general(
        w_ref[...], x_ref[...], (((1,), (1,)), ((), ())),
        preferred_element_type=jnp.float32)
    scores_t = jax.nn.sigmoid(logits_t)
    scores_ref[...] = scores_t.T

    biased_t = scores_t + bias_ref[...]
    iota_f = lax.broadcasted_iota(jnp.int32, (E, BN), 0).astype(jnp.float32)

    work = biased_t
    idx_rows = []
    raw_rows = []
    neg_inf = jnp.float32(-jnp.inf)
    for _ in range(K):
        m = jnp.max(work, axis=0, keepdims=True)
        masked_iota = jnp.where(work == m, iota_f, jnp.float32(E))
        idx_f = jnp.min(masked_iota, axis=0, keepdims=True)
        onehot = masked_iota == idx_f
        raw = jnp.sum(jnp.where(onehot, scores_t, 0.0), axis=0, keepdims=True)
        idx_rows.append(idx_f)
        raw_rows.append(raw)
        work = jnp.where(onehot, neg_inf, work)

    sel = (work == neg_inf).astype(jnp.float32)
    counts = jnp.sum(sel, axis=1, keepdims=True)

    idx_t = jnp.concatenate(idx_rows, axis=0)
    raw_t = jnp.concatenate(raw_rows, axis=0)
    denom = jnp.sum(raw_t, axis=0, keepdims=True) + 1e-20
    tops_t = raw_t / denom
    idx_ref[...] = idx_t.T.astype(jnp.int32)
    tops_ref[...] = tops_t.T

    ent_part = jnp.sum(tops_t * jnp.log(tops_t))

    @pl.when(i == 0)
    def _init():
        counts_ref[...] = counts
        ent_ref[...] = jnp.full((1, 1), ent_part, jnp.float32)

    @pl.when(i > 0)
    def _acc():
        counts_ref[...] += counts
        ent_ref[...] += ent_part

    @pl.when(i == nsteps - 1)
    def _fin():
        ent_ref[...] = -ent_ref[...] * (1.0 / N)


@jax.jit
def kernel(x, expert_bias, W):
    grid = (N // BN,)
    scores, tops, idx, counts, ent = pl.pallas_call(
        _router_body,
        grid=grid,
        in_specs=[
            pl.BlockSpec((BN, DIM), lambda i: (i, 0)),
            pl.BlockSpec((E, 1), lambda i: (0, 0)),
            pl.BlockSpec((E, DIM), lambda i: (0, 0)),
        ],
        out_specs=[
            pl.BlockSpec((BN, E), lambda i: (i, 0)),
            pl.BlockSpec((BN, K), lambda i: (i, 0)),
            pl.BlockSpec((BN, K), lambda i: (i, 0)),
            pl.BlockSpec((E, 1), lambda i: (0, 0)),
            pl.BlockSpec((1, 1), lambda i: (0, 0)),
        ],
        out_shape=[
            jax.ShapeDtypeStruct((N, E), jnp.float32),
            jax.ShapeDtypeStruct((N, K), jnp.float32),
            jax.ShapeDtypeStruct((N, K), jnp.int32),
            jax.ShapeDtypeStruct((E, 1), jnp.float32),
            jax.ShapeDtypeStruct((1, 1), jnp.float32),
        ],
    )(x, expert_bias.reshape(E, 1), W)
    return (tops, scores, idx, counts.reshape(E), ent.reshape(()))


# speedup vs baseline: 2.3863x; 1.1672x over previous
import functools

import jax
import jax.numpy as jnp
from jax import lax
from jax.experimental import pallas as pl
from jax.experimental.pallas import tpu as pltpu

N = 16384
DIM = 2048
E = 64
K = 8
BN = 1024


def _router_body(x_ref, bias_ref, w_ref, scores_ref, tops_ref, idx_ref,
                 counts_ref, ent_ref):
    i = pl.program_id(0)
    nsteps = pl.num_programs(0)

    logits_t = lax.dot_general(
        w_ref[...], x_ref[...], (((1,), (1,)), ((), ())),
        preferred_element_type=jnp.float32)
    scores_t = jax.nn.sigmoid(logits_t)
    scores_ref[...] = scores_t.T

    biased_t = scores_t + bias_ref[...]
    iota_f = lax.broadcasted_iota(jnp.int32, (E, BN), 0).astype(jnp.float32)

    work = biased_t
    idx_rows = []
    raw_rows = []
    neg_inf = jnp.float32(-jnp.inf)
    for _ in range(K):
        m = jnp.max(work, axis=0, keepdims=True)
        masked_iota = jnp.where(work == m, iota_f, jnp.float32(E))
        idx_f = jnp.min(masked_iota, axis=0, keepdims=True)
        onehot = masked_iota == idx_f
        raw = jnp.sum(jnp.where(onehot, scores_t, 0.0), axis=0, keepdims=True)
        idx_rows.append(idx_f)
        raw_rows.append(raw)
        work = jnp.where(onehot, neg_inf, work)

    sel = (work == neg_inf).astype(jnp.float32)
    counts = jnp.sum(sel, axis=1, keepdims=True)

    idx_t = jnp.concatenate(idx_rows, axis=0)
    raw_t = jnp.concatenate(raw_rows, axis=0)
    denom = jnp.sum(raw_t, axis=0, keepdims=True) + 1e-20
    tops_t = raw_t / denom
    idx_ref[...] = idx_t.T.astype(jnp.int32)
    tops_ref[...] = tops_t.T

    ent_part = jnp.sum(tops_t * jnp.log(tops_t))

    @pl.when(i == 0)
    def _init():
        counts_ref[...] = counts
        ent_ref[...] = jnp.full((1, 1), ent_part, jnp.float32)

    @pl.when(i > 0)
    def _acc():
        counts_ref[...] += counts
        ent_ref[...] += ent_part

    @pl.when(i == nsteps - 1)
    def _fin():
        ent_ref[...] = -ent_ref[...] * (1.0 / N)


@jax.jit
def kernel(x, expert_bias, W):
    grid = (N // BN,)
    scores, tops, idx, counts, ent = pl.pallas_call(
        _router_body,
        grid=grid,
        in_specs=[
            pl.BlockSpec((BN, DIM), lambda i: (i, 0)),
            pl.BlockSpec((E, 1), lambda i: (0, 0)),
            pl.BlockSpec((E, DIM), lambda i: (0, 0)),
        ],
        out_specs=[
            pl.BlockSpec((BN, E), lambda i: (i, 0)),
            pl.BlockSpec((BN, K), lambda i: (i, 0)),
            pl.BlockSpec((BN, K), lambda i: (i, 0)),
            pl.BlockSpec((E, 1), lambda i: (0, 0)),
            pl.BlockSpec((1, 1), lambda i: (0, 0)),
        ],
        out_shape=[
            jax.ShapeDtypeStruct((N, E), jnp.float32),
            jax.ShapeDtypeStruct((N, K), jnp.float32),
            jax.ShapeDtypeStruct((N, K), jnp.int32),
            jax.ShapeDtypeStruct((E, 1), jnp.float32),
            jax.ShapeDtypeStruct((1, 1), jnp.float32),
        ],
    )(x, expert_bias.reshape(E, 1), W)
    return (tops, scores, idx, counts.reshape(E), ent.reshape(()))


# speedup vs baseline: 2.4730x; 1.0363x over previous
import functools

import jax
import jax.numpy as jnp
from jax import lax
from jax.experimental import pallas as pl
from jax.experimental.pallas import tpu as pltpu

N = 16384
DIM = 2048
E = 64
K = 8
BN = 2048


def _router_body(x_ref, bias_ref, w_ref, scores_ref, tops_ref, idx_ref,
                 counts_ref, ent_ref):
    i = pl.program_id(0)
    nsteps = pl.num_programs(0)

    logits_t = lax.dot_general(
        w_ref[...], x_ref[...], (((1,), (1,)), ((), ())),
        preferred_element_type=jnp.float32)
    scores_t = jax.nn.sigmoid(logits_t)
    scores_ref[...] = scores_t.T

    biased_t = scores_t + bias_ref[...]
    iota_f = lax.broadcasted_iota(jnp.int32, (E, BN), 0).astype(jnp.float32)

    work = biased_t
    idx_rows = []
    raw_rows = []
    neg_inf = jnp.float32(-jnp.inf)
    for _ in range(K):
        m = jnp.max(work, axis=0, keepdims=True)
        masked_iota = jnp.where(work == m, iota_f, jnp.float32(E))
        idx_f = jnp.min(masked_iota, axis=0, keepdims=True)
        onehot = masked_iota == idx_f
        raw = jnp.sum(jnp.where(onehot, scores_t, 0.0), axis=0, keepdims=True)
        idx_rows.append(idx_f)
        raw_rows.append(raw)
        work = jnp.where(onehot, neg_inf, work)

    sel = (work == neg_inf).astype(jnp.float32)
    counts = jnp.sum(sel, axis=1, keepdims=True)

    idx_t = jnp.concatenate(idx_rows, axis=0)
    raw_t = jnp.concatenate(raw_rows, axis=0)
    denom = jnp.sum(raw_t, axis=0, keepdims=True) + 1e-20
    tops_t = raw_t / denom
    idx_ref[...] = idx_t.T.astype(jnp.int32)
    tops_ref[...] = tops_t.T

    ent_part = jnp.sum(tops_t * jnp.log(tops_t))

    @pl.when(i == 0)
    def _init():
        counts_ref[...] = counts
        ent_ref[...] = jnp.full((1, 1), ent_part, jnp.float32)

    @pl.when(i > 0)
    def _acc():
        counts_ref[...] += counts
        ent_ref[...] += ent_part

    @pl.when(i == nsteps - 1)
    def _fin():
        ent_ref[...] = -ent_ref[...] * (1.0 / N)


@jax.jit
def kernel(x, expert_bias, W):
    grid = (N // BN,)
    scores, tops, idx, counts, ent = pl.pallas_call(
        _router_body,
        grid=grid,
        in_specs=[
            pl.BlockSpec((BN, DIM), lambda i: (i, 0)),
            pl.BlockSpec((E, 1), lambda i: (0, 0)),
            pl.BlockSpec((E, DIM), lambda i: (0, 0)),
        ],
        out_specs=[
            pl.BlockSpec((BN, E), lambda i: (i, 0)),
            pl.BlockSpec((BN, K), lambda i: (i, 0)),
            pl.BlockSpec((BN, K), lambda i: (i, 0)),
            pl.BlockSpec((E, 1), lambda i: (0, 0)),
            pl.BlockSpec((1, 1), lambda i: (0, 0)),
        ],
        out_shape=[
            jax.ShapeDtypeStruct((N, E), jnp.float32),
            jax.ShapeDtypeStruct((N, K), jnp.float32),
            jax.ShapeDtypeStruct((N, K), jnp.int32),
            jax.ShapeDtypeStruct((E, 1), jnp.float32),
            jax.ShapeDtypeStruct((1, 1), jnp.float32),
        ],
    )(x, expert_bias.reshape(E, 1), W)
    return (tops, scores, idx, counts.reshape(E), ent.reshape(()))
